# Initial kernel scaffold; baseline (speedup 1.0000x reference)
import jax
import jax.numpy as jnp
from jax.experimental import pallas as pl


def kernel(order_ids, turn_ids, role_ids, turn_table):
    raise NotImplementedError("write your pallas kernel here")


# speedup vs baseline: 5.5606x; 5.5606x over previous
import functools

import jax
import jax.numpy as jnp
from jax import lax
from jax.experimental import pallas as pl
from jax.experimental.pallas import tpu as pltpu
from jax.experimental.pallas import tpu_sc as plsc

_NC = 2
_NS = 16
_NW = _NC * _NS

_CHUNK = 128


def _gather_grid(table, idx2d, n_chunks_per_w):
    V, D = table.shape
    total_rows = idx2d.shape[0] * _CHUNK
    mesh = plsc.VectorSubcoreMesh(core_axis_name="c", subcore_axis_name="s")

    @functools.partial(
        pl.kernel,
        out_type=jax.ShapeDtypeStruct((total_rows, D), jnp.float32),
        mesh=mesh,
        scratch_types=[
            pltpu.VMEM((n_chunks_per_w, _CHUNK), jnp.int32),
            pltpu.VMEM((_CHUNK, D), jnp.float32),
            pltpu.SemaphoreType.DMA,
        ],
    )
    def run(table_hbm, idx_hbm, out_hbm, idx_v, rows_v, sem):
        wid = lax.axis_index("s") * _NC + lax.axis_index("c")
        chunk_base = wid * n_chunks_per_w
        row_base = chunk_base * _CHUNK
        pltpu.sync_copy(idx_hbm.at[pl.ds(chunk_base, n_chunks_per_w)], idx_v)

        def step(j, carry):
            pltpu.async_copy(table_hbm.at[idx_v.at[j]], rows_v, sem).wait()
            pltpu.sync_copy(
                rows_v, out_hbm.at[pl.ds(row_base + j * _CHUNK, _CHUNK)]
            )
            return carry

        lax.fori_loop(0, n_chunks_per_w, step, 0)

    return run(table, idx2d)


def kernel(order_ids, turn_ids, role_ids, turn_table):
    B, S = turn_ids.shape
    V, D = turn_table.shape
    N = B * S
    assert N % (_NW * _CHUNK) == 0
    n_chunks_per_w = N // (_NW * _CHUNK)
    idx2d = turn_ids.reshape(N // _CHUNK, _CHUNK).astype(jnp.int32)
    out = _gather_grid(turn_table, idx2d, n_chunks_per_w)
    return out.reshape(B, S, D)


# speedup vs baseline: 6.3102x; 1.1348x over previous
import functools

import jax
import jax.numpy as jnp
from jax import lax
from jax.experimental import pallas as pl
from jax.experimental.pallas import tpu as pltpu
from jax.experimental.pallas import tpu_sc as plsc

_NC = 2
_NS = 16
_NW = _NC * _NS

_CHUNK = 128


_NBUF = 4


def _gather_grid(table, idx2d, n_chunks_per_w):
    V, D = table.shape
    total_rows = idx2d.shape[0] * _CHUNK
    assert n_chunks_per_w % _NBUF == 0
    n_groups = n_chunks_per_w // _NBUF
    mesh = plsc.VectorSubcoreMesh(core_axis_name="c", subcore_axis_name="s")

    @functools.partial(
        pl.kernel,
        out_type=jax.ShapeDtypeStruct((total_rows, D), jnp.float32),
        mesh=mesh,
        scratch_types=[
            pltpu.VMEM((n_chunks_per_w, _CHUNK), jnp.int32),
            [pltpu.VMEM((_CHUNK, D), jnp.float32) for _ in range(_NBUF)],
            [pltpu.SemaphoreType.DMA for _ in range(_NBUF)],
            [pltpu.SemaphoreType.DMA for _ in range(_NBUF)],
        ],
    )
    def run(table_hbm, idx_hbm, out_hbm, idx_v, bufs, gsems, osems):
        wid = lax.axis_index("s") * _NC + lax.axis_index("c")
        chunk_base = wid * n_chunks_per_w
        row_base = chunk_base * _CHUNK
        pltpu.sync_copy(idx_hbm.at[pl.ds(chunk_base, n_chunks_per_w)], idx_v)

        def out_slice(j):
            return out_hbm.at[pl.ds(row_base + j * _CHUNK, _CHUNK)]

        def step(g, carry):
            @pl.when(g > 0)
            def _():
                for b in range(_NBUF):
                    pltpu.make_async_copy(
                        bufs[b], out_slice(0), osems[b]
                    ).wait()

            descs = []
            for b in range(_NBUF):
                j = g * _NBUF + b
                descs.append(
                    pltpu.async_copy(
                        table_hbm.at[idx_v.at[j]], bufs[b], gsems[b]
                    )
                )
            for b in range(_NBUF):
                j = g * _NBUF + b
                descs[b].wait()
                pltpu.async_copy(bufs[b], out_slice(j), osems[b])
            return carry

        lax.fori_loop(0, n_groups, step, 0)
        for b in range(_NBUF):
            pltpu.make_async_copy(bufs[b], out_slice(0), osems[b]).wait()

    return run(table, idx2d)


def kernel(order_ids, turn_ids, role_ids, turn_table):
    B, S = turn_ids.shape
    V, D = turn_table.shape
    N = B * S
    assert N % (_NW * _CHUNK) == 0
    n_chunks_per_w = N // (_NW * _CHUNK)
    idx2d = turn_ids.reshape(N // _CHUNK, _CHUNK).astype(jnp.int32)
    out = _gather_grid(turn_table, idx2d, n_chunks_per_w)
    return out.reshape(B, S, D)


# speedup vs baseline: 12.8810x; 2.0413x over previous
import functools

import jax
import jax.numpy as jnp
from jax import lax
from jax.experimental import pallas as pl
from jax.experimental.pallas import tpu as pltpu
from jax.experimental.pallas import tpu_sc as plsc

_NC = 2
_NS = 16
_NW = _NC * _NS

_CHUNK = 128


_NBUF = 4


def _gather_grid(table, idx2d, n_chunks_per_w):
    V, D = table.shape
    total_rows = idx2d.shape[0] * _CHUNK
    assert n_chunks_per_w % _NBUF == 0
    n_groups = n_chunks_per_w // _NBUF
    mesh = plsc.VectorSubcoreMesh(core_axis_name="c", subcore_axis_name="s")

    @functools.partial(
        pl.kernel,
        out_type=jax.ShapeDtypeStruct((total_rows, D), jnp.float32),
        mesh=mesh,
        scratch_types=[
            pltpu.VMEM_SHARED((V, D), jnp.float32),
            pltpu.VMEM((n_chunks_per_w, _CHUNK), jnp.int32),
            [pltpu.VMEM((_CHUNK, D), jnp.float32) for _ in range(_NBUF)],
            [pltpu.SemaphoreType.DMA for _ in range(_NBUF)],
            [pltpu.SemaphoreType.DMA for _ in range(_NBUF)],
        ],
    )
    def run(table_hbm, idx_hbm, out_hbm, table_sh, idx_v, bufs, gsems, osems):
        wid = lax.axis_index("s") * _NC + lax.axis_index("c")
        chunk_base = wid * n_chunks_per_w
        row_base = chunk_base * _CHUNK
        @pl.when(lax.axis_index("s") == 0)
        def _():
            pltpu.sync_copy(table_hbm, table_sh)

        pltpu.sync_copy(idx_hbm.at[pl.ds(chunk_base, n_chunks_per_w)], idx_v)
        plsc.subcore_barrier()

        def out_slice(j):
            return out_hbm.at[pl.ds(row_base + j * _CHUNK, _CHUNK)]

        def step(g, carry):
            @pl.when(g > 0)
            def _():
                for b in range(_NBUF):
                    pltpu.make_async_copy(
                        bufs[b], out_slice(0), osems[b]
                    ).wait()

            descs = []
            for b in range(_NBUF):
                j = g * _NBUF + b
                descs.append(
                    pltpu.async_copy(
                        table_sh.at[idx_v.at[j]], bufs[b], gsems[b]
                    )
                )
            for b in range(_NBUF):
                j = g * _NBUF + b
                descs[b].wait()
                pltpu.async_copy(bufs[b], out_slice(j), osems[b])
            return carry

        lax.fori_loop(0, n_groups, step, 0)
        for b in range(_NBUF):
            pltpu.make_async_copy(bufs[b], out_slice(0), osems[b]).wait()

    return run(table, idx2d)


def kernel(order_ids, turn_ids, role_ids, turn_table):
    B, S = turn_ids.shape
    V, D = turn_table.shape
    N = B * S
    assert N % (_NW * _CHUNK) == 0
    n_chunks_per_w = N // (_NW * _CHUNK)
    idx2d = turn_ids.reshape(N // _CHUNK, _CHUNK).astype(jnp.int32)
    out = _gather_grid(turn_table, idx2d, n_chunks_per_w)
    return out.reshape(B, S, D)


# speedup vs baseline: 15.6908x; 1.2181x over previous
import functools

import jax
import jax.numpy as jnp
from jax import lax
from jax.experimental import pallas as pl
from jax.experimental.pallas import tpu as pltpu
from jax.experimental.pallas import tpu_sc as plsc

_NC = 2
_NS = 16
_NW = _NC * _NS

_CHUNK = 128


_NBUF = 4


def _gather_grid(table, idx2d, n_chunks_per_w):
    V, D = table.shape
    total_rows = idx2d.shape[0] * _CHUNK
    assert n_chunks_per_w % _NBUF == 0
    n_groups = n_chunks_per_w // _NBUF
    mesh = plsc.VectorSubcoreMesh(core_axis_name="c", subcore_axis_name="s")

    @functools.partial(
        pl.kernel,
        out_type=jax.ShapeDtypeStruct((total_rows, D), jnp.float32),
        mesh=mesh,
        scratch_types=[
            pltpu.VMEM_SHARED((V, D), jnp.float32),
            pltpu.VMEM((n_chunks_per_w, _CHUNK), jnp.int32),
            pltpu.VMEM((_NBUF * _CHUNK, D), jnp.float32),
            [pltpu.SemaphoreType.DMA for _ in range(_NBUF)],
            [pltpu.SemaphoreType.DMA for _ in range(2)],
        ],
    )
    def run(table_hbm, idx_hbm, out_hbm, table_sh, idx_v, big, gsems, osems):
        wid = lax.axis_index("s") * _NC + lax.axis_index("c")
        chunk_base = wid * n_chunks_per_w
        row_base = chunk_base * _CHUNK
        @pl.when(lax.axis_index("s") == 0)
        def _():
            pltpu.sync_copy(table_hbm, table_sh)

        pltpu.sync_copy(idx_hbm.at[pl.ds(chunk_base, n_chunks_per_w)], idx_v)
        plsc.subcore_barrier()

        half = _NBUF // 2
        hrows = half * _CHUNK

        def halfbuf(h):
            return big.at[pl.ds(h * hrows, hrows)]

        def out_half(g, h):
            start = row_base + (g * _NBUF + h * half) * _CHUNK
            return out_hbm.at[pl.ds(start, hrows)]

        def step(g, carry):
            descs = []
            for h in range(2):
                @pl.when(g > 0)
                def _():
                    pltpu.make_async_copy(
                        halfbuf(h), out_half(0, 0), osems[h]
                    ).wait()

                for i in range(half):
                    b = h * half + i
                    j = g * _NBUF + b
                    descs.append(
                        pltpu.async_copy(
                            table_sh.at[idx_v.at[j]],
                            big.at[pl.ds(b * _CHUNK, _CHUNK)],
                            gsems[b],
                        )
                    )
            for h in range(2):
                for i in range(half):
                    descs[h * half + i].wait()
                pltpu.async_copy(halfbuf(h), out_half(g, h), osems[h])
            return carry

        lax.fori_loop(0, n_groups, step, 0)
        for h in range(2):
            pltpu.make_async_copy(halfbuf(h), out_half(0, 0), osems[h]).wait()

    return run(table, idx2d)


def kernel(order_ids, turn_ids, role_ids, turn_table):
    B, S = turn_ids.shape
    V, D = turn_table.shape
    N = B * S
    assert N % (_NW * _CHUNK) == 0
    n_chunks_per_w = N // (_NW * _CHUNK)
    idx2d = turn_ids.reshape(N // _CHUNK, _CHUNK).astype(jnp.int32)
    out = _gather_grid(turn_table, idx2d, n_chunks_per_w)
    return out.reshape(B, S, D)


# speedup vs baseline: 15.8911x; 1.0128x over previous
import functools

import jax
import jax.numpy as jnp
from jax import lax
from jax.experimental import pallas as pl
from jax.experimental.pallas import tpu as pltpu
from jax.experimental.pallas import tpu_sc as plsc

_NC = 2
_NS = 16
_NW = _NC * _NS

_CHUNK = 128


_NSLOT = 3
_UCHUNK = 2


def _gather_grid(table, idx2d, n_chunks_per_w):
    V, D = table.shape
    total_rows = idx2d.shape[0] * _CHUNK
    urows = _UCHUNK * _CHUNK
    n_units = n_chunks_per_w // _UCHUNK
    assert n_units % _NSLOT == 1
    n_loop = n_units // _NSLOT
    mesh = plsc.VectorSubcoreMesh(core_axis_name="c", subcore_axis_name="s")

    @functools.partial(
        pl.kernel,
        out_type=jax.ShapeDtypeStruct((total_rows, D), jnp.float32),
        mesh=mesh,
        scratch_types=[
            pltpu.VMEM_SHARED((V, D), jnp.float32),
            pltpu.VMEM((_NSLOT * _UCHUNK, _CHUNK), jnp.int32),
            pltpu.VMEM((_NSLOT * urows, D), jnp.float32),
            [
                [pltpu.SemaphoreType.DMA for _ in range(_UCHUNK)]
                for _ in range(_NSLOT)
            ],
            [pltpu.SemaphoreType.DMA for _ in range(_NSLOT)],
            [pltpu.SemaphoreType.DMA for _ in range(_NSLOT)],
        ],
    )
    def run(
        table_hbm, idx_hbm, out_hbm, table_sh, idx_v, big, gsems, osems, isems
    ):
        wid = lax.axis_index("s") * _NC + lax.axis_index("c")
        chunk_base = wid * n_chunks_per_w
        row_base = chunk_base * _CHUNK
        @pl.when(lax.axis_index("s") == 0)
        def _():
            pltpu.sync_copy(table_hbm, table_sh)

        def slotbuf(s):
            return big.at[pl.ds(s * urows, urows)]

        def out_unit(u):
            return out_hbm.at[pl.ds(row_base + u * urows, urows)]

        def drain_out(s):
            pltpu.make_async_copy(slotbuf(s), out_unit(0), osems[s]).wait()

        def idx_slot(s):
            return idx_v.at[pl.ds(s * _UCHUNK, _UCHUNK)]

        def fetch_idx(u, s):
            pltpu.async_copy(
                idx_hbm.at[pl.ds(chunk_base + u * _UCHUNK, _UCHUNK)],
                idx_slot(s),
                isems[s],
            )

        def wait_idx(s):
            pltpu.make_async_copy(
                idx_hbm.at[pl.ds(0, _UCHUNK)], idx_slot(s), isems[s]
            ).wait()

        def start_gathers(s):
            descs = []
            for i in range(_UCHUNK):
                descs.append(
                    pltpu.async_copy(
                        table_sh.at[idx_v.at[s * _UCHUNK + i]],
                        big.at[pl.ds(s * urows + i * _CHUNK, _CHUNK)],
                        gsems[s][i],
                    )
                )
            return descs

        for s in range(_NSLOT):
            fetch_idx(s, s)
        plsc.subcore_barrier()

        def step(g, carry):
            descs = []
            for s in range(_NSLOT):
                u = g * _NSLOT + s

                @pl.when(g > 0)
                def _():
                    drain_out(s)

                wait_idx(s)
                descs.append(start_gathers(s))
            for s in range(_NSLOT):
                u = g * _NSLOT + s
                for d in descs[s]:
                    d.wait()
                pltpu.async_copy(slotbuf(s), out_unit(u), osems[s])

                @pl.when(u + _NSLOT < n_units)
                def _():
                    fetch_idx(u + _NSLOT, s)

            return carry

        lax.fori_loop(0, n_loop, step, 0)

        u_last = n_loop * _NSLOT
        drain_out(0)
        wait_idx(0)
        for d in start_gathers(0):
            d.wait()
        pltpu.async_copy(slotbuf(0), out_unit(u_last), osems[0])
        for s in range(_NSLOT):
            drain_out(s)

    return run(table, idx2d)


def kernel(order_ids, turn_ids, role_ids, turn_table):
    B, S = turn_ids.shape
    V, D = turn_table.shape
    N = B * S
    assert N % (_NW * _CHUNK) == 0
    n_chunks_per_w = N // (_NW * _CHUNK)
    idx2d = turn_ids.reshape(N // _CHUNK, _CHUNK).astype(jnp.int32)
    out = _gather_grid(turn_table, idx2d, n_chunks_per_w)
    return out.reshape(B, S, D)
